# Initial kernel scaffold; baseline (speedup 1.0000x reference)
import jax
import jax.numpy as jnp
from jax.experimental import pallas as pl


def kernel(x, edge_index, edge_weight, W1, b1, W2, b2):
    raise NotImplementedError("write your pallas kernel here")


# speedup vs baseline: 37.9610x; 37.9610x over previous
import functools

import jax
import jax.numpy as jnp
from jax import lax
from jax.experimental import pallas as pl
from jax.experimental.pallas import tpu as pltpu
from jax.experimental.pallas import tpu_sc as plsc

_N = 10000
_E = 320000
_D = 128
_H = 16
_C = 10

_NW = 32
_CH = 1024
_RB = _CH // 128
_PW = 10240
_EP = _NW * _PW
_NCH = _PW // _CH
_NP = 10240
_NPS = _NP // 16

_mesh = plsc.VectorSubcoreMesh(core_axis_name="c", subcore_axis_name="s")


@functools.partial(
    pl.kernel,
    out_type=jax.ShapeDtypeStruct((2 * _NP,), jnp.float32),
    scratch_types=[
        pltpu.VMEM((_RB, 128), jnp.int32),
        pltpu.VMEM((_CH,), jnp.float32),
        pltpu.VMEM((_NPS,), jnp.float32),
        pltpu.VMEM_SHARED((_NP,), jnp.float32),
    ],
    mesh=_mesh,
    compiler_params=pltpu.CompilerParams(use_tc_tiling_on_sc=False),
)
def _deg_kernel(dst_hbm, w_hbm, out_hbm, dst_ref, w_ref, zb, deg_sh):
    c = lax.axis_index("c")
    s = lax.axis_index("s")
    wid = c * 16 + s

    def zrow(i, carry):
        zb[pl.ds(i * 16, 16)] = jnp.zeros((16,), jnp.float32)
        return carry

    lax.fori_loop(0, _NPS // 16, zrow, 0)
    pltpu.sync_copy(zb, deg_sh.at[pl.ds(s * _NPS, _NPS)])
    plsc.subcore_barrier()

    def chunk(ci, carry):
        e0 = wid * _PW + ci * _CH
        r0 = wid * (_PW // 128) + ci * _RB
        pltpu.sync_copy(dst_hbm.at[pl.ds(r0, _RB)], dst_ref)
        pltpu.sync_copy(w_hbm.at[pl.ds(e0, _CH)], w_ref)
        for j in range(_RB):
            pltpu.sync_copy(
                w_ref.at[pl.ds(j * 128, 128)],
                deg_sh.at[dst_ref.at[j]],
                add=True,
            )
        return carry

    lax.fori_loop(0, _NCH, chunk, 0)
    plsc.subcore_barrier()
    pltpu.sync_copy(
        deg_sh.at[pl.ds(s * _NPS, _NPS)],
        out_hbm.at[pl.ds(c * _NP + s * _NPS, _NPS)],
    )


@functools.partial(
    pl.kernel,
    out_type=jax.ShapeDtypeStruct((2 * _NP, _H), jnp.float32),
    scratch_types=[
        pltpu.VMEM((_RB, 128), jnp.int32),
        pltpu.VMEM((_RB, 128), jnp.int32),
        pltpu.VMEM((_CH,), jnp.float32),
        pltpu.VMEM((_CH, _H), jnp.float32),
        pltpu.VMEM_SHARED((_NP, _H), jnp.float32),
        pltpu.SemaphoreType.DMA,
    ],
    mesh=_mesh,
    compiler_params=pltpu.CompilerParams(use_tc_tiling_on_sc=False),
)
def _agg_kernel(tab_hbm, src_hbm, dst_hbm, w_hbm, out_hbm,
                src_ref, dst_ref, w_ref, rows_ref, acc_sh, sem):
    c = lax.axis_index("c")
    s = lax.axis_index("s")
    wid = c * 16 + s

    def zrow(i, carry):
        rows_ref[i, :] = jnp.zeros((_H,), jnp.float32)
        return carry

    lax.fori_loop(0, _NPS, zrow, 0)
    pltpu.sync_copy(rows_ref.at[pl.ds(0, _NPS)], acc_sh.at[pl.ds(s * _NPS, _NPS)])
    plsc.subcore_barrier()

    def chunk(ci, carry):
        e0 = wid * _PW + ci * _CH
        r0 = wid * (_PW // 128) + ci * _RB
        pltpu.sync_copy(src_hbm.at[pl.ds(r0, _RB)], src_ref)
        pltpu.sync_copy(dst_hbm.at[pl.ds(r0, _RB)], dst_ref)
        pltpu.sync_copy(w_hbm.at[pl.ds(e0, _CH)], w_ref)
        cps = [
            pltpu.async_copy(
                tab_hbm.at[src_ref.at[j]],
                rows_ref.at[pl.ds(j * 128, 128)],
                sem,
            )
            for j in range(_RB)
        ]
        for cp in cps:
            cp.wait()

        def scale(i, carry2):
            wv = w_ref[pl.ds(i * 16, 16)]
            e = i * 16
            for k in range(16):
                rows_ref[e + k, :] = rows_ref[e + k, :] * wv[k]
            return carry2

        lax.fori_loop(0, _CH // 16, scale, 0)
        for j in range(_RB):
            pltpu.sync_copy(
                rows_ref.at[pl.ds(j * 128, 128)],
                acc_sh.at[dst_ref.at[j]],
                add=True,
            )
        return carry

    lax.fori_loop(0, _NCH, chunk, 0)
    plsc.subcore_barrier()
    pltpu.sync_copy(
        acc_sh.at[pl.ds(s * _NPS, _NPS)],
        out_hbm.at[pl.ds(c * _NP + s * _NPS, _NPS)],
    )


_BN = 2000


def _tcb_body(deg_ref, x_ref, w1_ref, hp_ref, dinv_ref):
    deg = deg_ref[:, 0:1] + deg_ref[:, 1:2] + 1.0
    dinv = jnp.where(deg > 0, lax.rsqrt(jnp.maximum(deg, 1e-12)), 0.0)
    h = jnp.dot(x_ref[:, :], w1_ref[:, :], preferred_element_type=jnp.float32)
    hp_ref[:, :] = h * dinv
    dinv_ref[:, :] = dinv


_tc_b = pl.pallas_call(
    _tcb_body,
    grid=(_N // _BN,),
    in_specs=[
        pl.BlockSpec((_BN, 2), lambda i: (i, 0)),
        pl.BlockSpec((_BN, _D), lambda i: (i, 0)),
        pl.BlockSpec((_D, _H), lambda i: (0, 0)),
    ],
    out_specs=[
        pl.BlockSpec((_BN, _H), lambda i: (i, 0)),
        pl.BlockSpec((_BN, 1), lambda i: (i, 0)),
    ],
    out_shape=[
        jax.ShapeDtypeStruct((_N, _H), jnp.float32),
        jax.ShapeDtypeStruct((_N, 1), jnp.float32),
    ],
)


def _tcd_body(a0_ref, a1_ref, hp_ref, dv_ref, b1_ref, w2_ref, x1_ref, gp_ref):
    dinv = dv_ref[:, :]
    x1 = dinv * (a0_ref[:, :] + a1_ref[:, :] + hp_ref[:, :]) + b1_ref[:, :]
    x1_ref[:, :] = x1
    r = jnp.maximum(x1, 0.0)
    g = jnp.dot(r, w2_ref[:, :], preferred_element_type=jnp.float32)
    gp_ref[:, :] = g * dinv


_tc_d = pl.pallas_call(
    _tcd_body,
    grid=(_N // _BN,),
    in_specs=[
        pl.BlockSpec((_BN, _H), lambda i: (i, 0)),
        pl.BlockSpec((_BN, _H), lambda i: (i, 0)),
        pl.BlockSpec((_BN, _H), lambda i: (i, 0)),
        pl.BlockSpec((_BN, 1), lambda i: (i, 0)),
        pl.BlockSpec((1, _H), lambda i: (0, 0)),
        pl.BlockSpec((_H, _H), lambda i: (0, 0)),
    ],
    out_specs=[
        pl.BlockSpec((_BN, _H), lambda i: (i, 0)),
        pl.BlockSpec((_BN, _H), lambda i: (i, 0)),
    ],
    out_shape=[
        jax.ShapeDtypeStruct((_N, _H), jnp.float32),
        jax.ShapeDtypeStruct((_N, _H), jnp.float32),
    ],
)


def _tcf_body(a0_ref, a1_ref, gp_ref, dv_ref, b2_ref, out_ref):
    o = dv_ref[:, :] * (a0_ref[:, :] + a1_ref[:, :] + gp_ref[:, :]) + b2_ref[:, :]
    mask = lax.broadcasted_iota(jnp.int32, (_BN, _H), 1) < _C
    z = jnp.where(mask, o, -3.0e38)
    m = jnp.max(z, axis=1, keepdims=True)
    e = jnp.where(mask, jnp.exp(z - m), 0.0)
    lse = jnp.log(jnp.sum(e, axis=1, keepdims=True)) + m
    out_ref[:, :] = o - lse


_tc_f = pl.pallas_call(
    _tcf_body,
    grid=(_N // _BN,),
    in_specs=[
        pl.BlockSpec((_BN, _H), lambda i: (i, 0)),
        pl.BlockSpec((_BN, _H), lambda i: (i, 0)),
        pl.BlockSpec((_BN, _H), lambda i: (i, 0)),
        pl.BlockSpec((_BN, 1), lambda i: (i, 0)),
        pl.BlockSpec((1, _H), lambda i: (0, 0)),
    ],
    out_specs=pl.BlockSpec((_BN, _H), lambda i: (i, 0)),
    out_shape=jax.ShapeDtypeStruct((_N, _H), jnp.float32),
)


def kernel(x, edge_index, edge_weight, W1, b1, W2, b2):
    src = edge_index[0].astype(jnp.int32)
    dst = edge_index[1].astype(jnp.int32)
    w = edge_weight.astype(jnp.float32)

    pad = _EP - _E
    fill = (jnp.arange(pad, dtype=jnp.int32) * 13) % _N
    srcp = jnp.concatenate([src, fill]).reshape(_EP // 128, 128)
    dstp = jnp.concatenate([dst, fill]).reshape(_EP // 128, 128)
    wp = jnp.concatenate([w, jnp.zeros((pad,), jnp.float32)])

    degp = _deg_kernel(dstp, wp)
    deg2 = jnp.stack([degp[:_N], degp[_NP:_NP + _N]], axis=1)

    hp, dinv = _tc_b(deg2, x, W1)

    acc1 = _agg_kernel(hp, srcp, dstp, wp)
    a0 = acc1[:_N]
    a1 = acc1[_NP:_NP + _N]

    W2p = jnp.zeros((_H, _H), jnp.float32).at[:, :_C].set(W2)
    x1, gp = _tc_d(a0, a1, hp, dinv, b1.reshape(1, _H), W2p)

    acc2 = _agg_kernel(gp, srcp, dstp, wp)
    c0 = acc2[:_N]
    c1 = acc2[_NP:_NP + _N]

    b2p = jnp.zeros((1, _H), jnp.float32).at[0, :_C].set(b2)
    out16 = _tc_f(c0, c1, gp, dinv, b2p)
    return (out16[:, :_C], x1)


# speedup vs baseline: 42.8868x; 1.1298x over previous
import functools

import jax
import jax.numpy as jnp
from jax import lax
from jax.experimental import pallas as pl
from jax.experimental.pallas import tpu as pltpu
from jax.experimental.pallas import tpu_sc as plsc

_N = 10000
_E = 320000
_D = 128
_H = 16
_C = 10

_NW = 32
_CH = 1024
_RB = _CH // 128
_PW = 10240
_EP = _NW * _PW
_NCH = _PW // _CH
_NP = 10240
_NPS = _NP // 16

_mesh = plsc.VectorSubcoreMesh(core_axis_name="c", subcore_axis_name="s")


@functools.partial(
    pl.kernel,
    out_type=jax.ShapeDtypeStruct((2 * _NP,), jnp.float32),
    scratch_types=[
        pltpu.VMEM((_RB, 128), jnp.int32),
        pltpu.VMEM((_CH,), jnp.float32),
        pltpu.VMEM((_NPS,), jnp.float32),
        pltpu.VMEM_SHARED((_NP,), jnp.float32),
    ],
    mesh=_mesh,
    compiler_params=pltpu.CompilerParams(use_tc_tiling_on_sc=False),
)
def _deg_kernel(dst_hbm, w_hbm, out_hbm, dst_ref, w_ref, zb, deg_sh):
    c = lax.axis_index("c")
    s = lax.axis_index("s")
    wid = c * 16 + s

    def zrow(i, carry):
        zb[pl.ds(i * 16, 16)] = jnp.zeros((16,), jnp.float32)
        return carry

    lax.fori_loop(0, _NPS // 16, zrow, 0)
    pltpu.sync_copy(zb, deg_sh.at[pl.ds(s * _NPS, _NPS)])
    plsc.subcore_barrier()

    def chunk(ci, carry):
        e0 = wid * _PW + ci * _CH
        r0 = wid * (_PW // 128) + ci * _RB
        pltpu.sync_copy(dst_hbm.at[pl.ds(r0, _RB)], dst_ref)
        pltpu.sync_copy(w_hbm.at[pl.ds(e0, _CH)], w_ref)
        for j in range(_RB):
            pltpu.sync_copy(
                w_ref.at[pl.ds(j * 128, 128)],
                deg_sh.at[dst_ref.at[j]],
                add=True,
            )
        return carry

    lax.fori_loop(0, _NCH, chunk, 0)
    plsc.subcore_barrier()
    pltpu.sync_copy(
        deg_sh.at[pl.ds(s * _NPS, _NPS)],
        out_hbm.at[pl.ds(c * _NP + s * _NPS, _NPS)],
    )


_NRS = _N // 16


@functools.partial(
    pl.kernel,
    out_type=[
        jax.ShapeDtypeStruct((_N, _H), jnp.float32),
        jax.ShapeDtypeStruct((_N, _H), jnp.float32),
    ],
    scratch_types=[
        pltpu.VMEM((2, _RB, 128), jnp.int32),
        pltpu.VMEM((2, _RB, 128), jnp.int32),
        pltpu.VMEM((2, _CH), jnp.float32),
        pltpu.VMEM((2, _CH, _H), jnp.float32),
        pltpu.VMEM_SHARED((_N, _H), jnp.float32),
        pltpu.SemaphoreType.DMA((2,)),
        pltpu.SemaphoreType.DMA((2,)),
    ],
    mesh=_mesh,
    compiler_params=pltpu.CompilerParams(use_tc_tiling_on_sc=False),
)
def _agg_kernel(tab_hbm, src_hbm, dst_hbm, w_hbm, out0_hbm, out1_hbm,
                src_ref, dst_ref, w_ref, rows_ref, acc_sh, gsem, ssem):
    c = lax.axis_index("c")
    s = lax.axis_index("s")
    wid = c * 16 + s

    def zrow(i, carry):
        rows_ref[0, i, :] = jnp.zeros((_H,), jnp.float32)
        return carry

    lax.fori_loop(0, _NRS, zrow, 0)
    pltpu.sync_copy(rows_ref.at[0, pl.ds(0, _NRS)],
                    acc_sh.at[pl.ds(s * _NRS, _NRS)])
    plsc.subcore_barrier()

    def stage_and_gather(ci, b):
        e0 = wid * _PW + ci * _CH
        r0 = wid * (_PW // 128) + ci * _RB
        pltpu.sync_copy(src_hbm.at[pl.ds(r0, _RB)], src_ref.at[b])
        pltpu.sync_copy(dst_hbm.at[pl.ds(r0, _RB)], dst_ref.at[b])
        pltpu.sync_copy(w_hbm.at[pl.ds(e0, _CH)], w_ref.at[b])
        return [
            pltpu.async_copy(
                tab_hbm.at[src_ref.at[b, j]],
                rows_ref.at[b, pl.ds(j * 128, 128)],
                gsem.at[b],
            )
            for j in range(_RB)
        ]

    gather_cps = {0: stage_and_gather(0, 0)}
    scatter_cps = {}
    for ci in range(_NCH):
        b = ci % 2
        for cp in gather_cps.pop(ci):
            cp.wait()

        def scale(i, carry2, b=b):
            wv = w_ref[b, pl.ds(i * 16, 16)]
            e = i * 16
            for k in range(16):
                rows_ref[b, e + k, :] = rows_ref[b, e + k, :] * wv[k]
            return carry2

        lax.fori_loop(0, _CH // 16, scale, 0)
        scatter_cps[ci] = [
            pltpu.async_copy(
                rows_ref.at[b, pl.ds(j * 128, 128)],
                acc_sh.at[dst_ref.at[b, j]],
                ssem.at[b],
                add=True,
            )
            for j in range(_RB)
        ]
        if ci + 1 < _NCH:
            if ci - 1 >= 0:
                for cp in scatter_cps.pop(ci - 1):
                    cp.wait()
            gather_cps[ci + 1] = stage_and_gather(ci + 1, 1 - b)
    for ci in sorted(scatter_cps):
        for cp in scatter_cps[ci]:
            cp.wait()
    plsc.subcore_barrier()

    @pl.when(c == 0)
    def _():
        pltpu.sync_copy(acc_sh.at[pl.ds(s * _NRS, _NRS)],
                        out0_hbm.at[pl.ds(s * _NRS, _NRS)])

    @pl.when(c == 1)
    def _():
        pltpu.sync_copy(acc_sh.at[pl.ds(s * _NRS, _NRS)],
                        out1_hbm.at[pl.ds(s * _NRS, _NRS)])


_BN = 2000


def _tcb_body(deg_ref, x_ref, w1_ref, hp_ref, dinv_ref):
    deg = deg_ref[:, 0:1] + deg_ref[:, 1:2] + 1.0
    dinv = jnp.where(deg > 0, lax.rsqrt(jnp.maximum(deg, 1e-12)), 0.0)
    h = jnp.dot(x_ref[:, :], w1_ref[:, :], preferred_element_type=jnp.float32)
    hp_ref[:, :] = h * dinv
    dinv_ref[:, :] = dinv


_tc_b = pl.pallas_call(
    _tcb_body,
    grid=(_N // _BN,),
    in_specs=[
        pl.BlockSpec((_BN, 2), lambda i: (i, 0)),
        pl.BlockSpec((_BN, _D), lambda i: (i, 0)),
        pl.BlockSpec((_D, _H), lambda i: (0, 0)),
    ],
    out_specs=[
        pl.BlockSpec((_BN, _H), lambda i: (i, 0)),
        pl.BlockSpec((_BN, 1), lambda i: (i, 0)),
    ],
    out_shape=[
        jax.ShapeDtypeStruct((_N, _H), jnp.float32),
        jax.ShapeDtypeStruct((_N, 1), jnp.float32),
    ],
)


def _tcd_body(a0_ref, a1_ref, hp_ref, dv_ref, b1_ref, w2_ref, x1_ref, gp_ref):
    dinv = dv_ref[:, :]
    x1 = dinv * (a0_ref[:, :] + a1_ref[:, :] + hp_ref[:, :]) + b1_ref[:, :]
    x1_ref[:, :] = x1
    r = jnp.maximum(x1, 0.0)
    g = jnp.dot(r, w2_ref[:, :], preferred_element_type=jnp.float32)
    gp_ref[:, :] = g * dinv


_tc_d = pl.pallas_call(
    _tcd_body,
    grid=(_N // _BN,),
    in_specs=[
        pl.BlockSpec((_BN, _H), lambda i: (i, 0)),
        pl.BlockSpec((_BN, _H), lambda i: (i, 0)),
        pl.BlockSpec((_BN, _H), lambda i: (i, 0)),
        pl.BlockSpec((_BN, 1), lambda i: (i, 0)),
        pl.BlockSpec((1, _H), lambda i: (0, 0)),
        pl.BlockSpec((_H, _H), lambda i: (0, 0)),
    ],
    out_specs=[
        pl.BlockSpec((_BN, _H), lambda i: (i, 0)),
        pl.BlockSpec((_BN, _H), lambda i: (i, 0)),
    ],
    out_shape=[
        jax.ShapeDtypeStruct((_N, _H), jnp.float32),
        jax.ShapeDtypeStruct((_N, _H), jnp.float32),
    ],
)


def _tcf_body(a0_ref, a1_ref, gp_ref, dv_ref, b2_ref, out_ref):
    o = dv_ref[:, :] * (a0_ref[:, :] + a1_ref[:, :] + gp_ref[:, :]) + b2_ref[:, :]
    mask = lax.broadcasted_iota(jnp.int32, (_BN, _H), 1) < _C
    z = jnp.where(mask, o, -3.0e38)
    m = jnp.max(z, axis=1, keepdims=True)
    e = jnp.where(mask, jnp.exp(z - m), 0.0)
    lse = jnp.log(jnp.sum(e, axis=1, keepdims=True)) + m
    out_ref[:, :] = lax.slice(o - lse, (0, 0), (_BN, _C))


_tc_f = pl.pallas_call(
    _tcf_body,
    grid=(_N // _BN,),
    in_specs=[
        pl.BlockSpec((_BN, _H), lambda i: (i, 0)),
        pl.BlockSpec((_BN, _H), lambda i: (i, 0)),
        pl.BlockSpec((_BN, _H), lambda i: (i, 0)),
        pl.BlockSpec((_BN, 1), lambda i: (i, 0)),
        pl.BlockSpec((1, _H), lambda i: (0, 0)),
    ],
    out_specs=pl.BlockSpec((_BN, _C), lambda i: (i, 0)),
    out_shape=jax.ShapeDtypeStruct((_N, _C), jnp.float32),
)


def kernel(x, edge_index, edge_weight, W1, b1, W2, b2):
    src = edge_index[0].astype(jnp.int32)
    dst = edge_index[1].astype(jnp.int32)
    w = edge_weight.astype(jnp.float32)

    pad = _EP - _E
    fill = (jnp.arange(pad, dtype=jnp.int32) * 13) % _N
    srcp = jnp.concatenate([src, fill]).reshape(_EP // 128, 128)
    dstp = jnp.concatenate([dst, fill]).reshape(_EP // 128, 128)
    wp = jnp.concatenate([w, jnp.zeros((pad,), jnp.float32)])

    degp = _deg_kernel(dstp, wp)
    deg2 = jnp.stack([degp[:_N], degp[_NP:_NP + _N]], axis=1)

    hp, dinv = _tc_b(deg2, x, W1)

    a0, a1 = _agg_kernel(hp, srcp, dstp, wp)

    W2p = jnp.zeros((_H, _H), jnp.float32).at[:, :_C].set(W2)
    x1, gp = _tc_d(a0, a1, hp, dinv, b1.reshape(1, _H), W2p)

    c0, c1 = _agg_kernel(gp, srcp, dstp, wp)

    b2p = jnp.zeros((1, _H), jnp.float32).at[0, :_C].set(b2)
    out = _tc_f(c0, c1, gp, dinv, b2p)
    return (out, x1)
